# Initial kernel scaffold; baseline (speedup 1.0000x reference)
import jax
import jax.numpy as jnp
from jax.experimental import pallas as pl


def kernel(graph_x, global_x, edge_index, batch, atom_W, atom_b, gin0_W, gin0_b, gin1_W, gin1_b, norm_g, norm_b, gate_W, gate_b, pool_W, pool_b, glob_W, glob_b, glob_g, glob_beta, emb_W, emb_b, h1_W, h1_b, h2_W, h2_b):
    raise NotImplementedError("write your pallas kernel here")


# speedup vs baseline: 4.6610x; 4.6610x over previous
import functools

import jax
import jax.numpy as jnp
from jax import lax
from jax.experimental import pallas as pl
from jax.experimental.pallas import tpu as pltpu
from jax.experimental.pallas import tpu_sc as plsc

_INFO = plsc.get_sparse_core_info()
_NC = _INFO.num_cores
_NS = _INFO.num_subcores
_NW = _NC * _NS
_CH = 128


def _pad_rows(n):
    per_tile = -(-(n + 1) // (_CH * _NS)) * _CH
    return per_tile * _NS, per_tile


def _make_edge_segsum(n, h, nch):
    npad, rows_per_tile = _pad_rows(n)
    zrep = rows_per_tile // _CH
    mesh = plsc.VectorSubcoreMesh(core_axis_name="c", subcore_axis_name="s")

    @functools.partial(
        pl.kernel,
        mesh=mesh,
        out_type=jax.ShapeDtypeStruct((_NC, npad, h), jnp.float32),
        scratch_types=[
            pltpu.VMEM((nch, _CH), jnp.int32),
            pltpu.VMEM((nch, _CH), jnp.int32),
            pltpu.VMEM((_CH, h), jnp.float32),
            pltpu.SemaphoreType.DMA,
            pltpu.VMEM_SHARED((npad, h), jnp.float32),
        ],
    )
    def seg(x_hbm, src_hbm, dst_hbm, out_hbm, sidx, didx, rows, sem, acc):
        c = lax.axis_index("c")
        s = lax.axis_index("s")
        wid = s * _NC + c

        pltpu.sync_copy(src_hbm.at[wid], sidx)
        pltpu.sync_copy(dst_hbm.at[wid], didx)

        def _zero(i, carry):
            for l in range(h // 16):
                rows[i, pl.ds(l * 16, 16)] = jnp.zeros((16,), jnp.float32)
            return carry

        lax.fori_loop(0, _CH, _zero, 0)
        for j in range(zrep):
            pltpu.sync_copy(rows, acc.at[pl.ds(s * rows_per_tile + j * _CH, _CH)])
        plsc.subcore_barrier()

        def _body(t, carry):
            pltpu.async_copy(x_hbm.at[sidx.at[t]], rows, sem).wait()
            pltpu.sync_copy(rows, acc.at[didx.at[t]], add=True)
            return carry

        lax.fori_loop(0, nch, _body, 0)
        plsc.subcore_barrier()
        pltpu.sync_copy(
            acc.at[pl.ds(s * rows_per_tile, rows_per_tile)],
            out_hbm.at[c, pl.ds(s * rows_per_tile, rows_per_tile)],
        )

    return seg


def _atom_body(x_ref, w_ref, b_ref, o_ref):
    o_ref[...] = jnp.maximum(x_ref[...] @ w_ref[...] + b_ref[...], 0.0)


def _gin_body(x_ref, a0_ref, a1_ref, w_ref, b_ref, o_ref):
    x = x_ref[...]
    t = x + a0_ref[...] + a1_ref[...]
    o_ref[...] = x + jnp.maximum(t @ w_ref[...] + b_ref[...], 0.0)


def _pool_body(x_ref, batch_ref, ng_ref, nb_ref, gw_ref, gb_ref, pw_ref,
               pb_ref, gf_ref, *, nb_graphs):
    x = x_ref[...]
    m = jnp.mean(x, axis=1, keepdims=True)
    v = jnp.mean((x - m) * (x - m), axis=1, keepdims=True)
    xln = (x - m) * lax.rsqrt(v + 1e-5) * ng_ref[...] + nb_ref[...]
    gate = jnp.sum(xln * gw_ref[...], axis=1, keepdims=True) + gb_ref[0, 0]
    xn = jnp.maximum(xln @ pw_ref[...] + pb_ref[...], 0.0)
    onehot = (batch_ref[...] ==
              lax.broadcasted_iota(jnp.int32, (1, nb_graphs), 1))
    oh = onehot.astype(jnp.float32)
    gmax = jnp.max(jnp.where(onehot, gate, -1e30), axis=0, keepdims=True)
    gmax_n = jnp.sum(oh * gmax, axis=1, keepdims=True)
    gexp = jnp.exp(gate - gmax_n)
    gsum = jnp.sum(oh * gexp, axis=0, keepdims=True)
    gsum_n = jnp.sum(oh * gsum, axis=1, keepdims=True)
    att = gexp / (gsum_n + 1e-16)
    gf_ref[...] = lax.dot_general(oh * att, xn, (((0,), (0,)), ((), ())),
                                  preferred_element_type=jnp.float32)


def _tail_body(gf_ref, gx_ref, gw_ref, gb_ref, gg_ref, gbeta_ref, ew_ref,
               eb_ref, h1w_ref, h1b_ref, h2w_ref, h2b_ref, o_ref):
    g = gx_ref[...] @ gw_ref[...] + gb_ref[...]
    m = jnp.mean(g, axis=1, keepdims=True)
    v = jnp.mean((g - m) * (g - m), axis=1, keepdims=True)
    gfeat = jnp.maximum(
        (g - m) * lax.rsqrt(v + 1e-5) * gg_ref[...] + gbeta_ref[...], 0.0)
    gf = gf_ref[...]
    merged = jnp.concatenate([gf, gfeat, gf * gfeat], axis=1)
    emb = merged @ ew_ref[...] + eb_ref[...]
    hid = jnp.maximum(emb @ h1w_ref[...] + h1b_ref[...], 0.0)
    o_ref[...] = jnp.sum(hid * h2w_ref[...], axis=1, keepdims=True) + h2b_ref[0, 0]


def kernel(graph_x, global_x, edge_index, batch, atom_W, atom_b, gin0_W,
           gin0_b, gin1_W, gin1_b, norm_g, norm_b, gate_W, gate_b, pool_W,
           pool_b, glob_W, glob_b, glob_g, glob_beta, emb_W, emb_b, h1_W,
           h1_b, h2_W, h2_b):
    n, ad = graph_x.shape
    h = atom_W.shape[1]
    e = edge_index.shape[1]
    b_graphs, gd = global_x.shape
    nch = -(-e // (_NW * _CH))
    epad = _NW * nch * _CH - e
    npad, _ = _pad_rows(n)

    src_r = jnp.concatenate(
        [edge_index[0].astype(jnp.int32),
         jnp.zeros((epad,), jnp.int32)]).reshape(_NW, nch, _CH)
    dst_r = jnp.concatenate(
        [edge_index[1].astype(jnp.int32),
         jnp.full((epad,), npad - 1, jnp.int32)]).reshape(_NW, nch, _CH)
    batch_2d = batch.astype(jnp.int32).reshape(n, 1)

    segsum = _make_edge_segsum(n, h, nch)

    blk = 1000
    grid = n // blk
    row_spec = pl.BlockSpec((blk, h), lambda i: (i, 0))
    full = lambda shape: pl.BlockSpec(shape, lambda i: tuple(0 for _ in shape))

    atom = pl.pallas_call(
        _atom_body,
        grid=(grid,),
        in_specs=[pl.BlockSpec((blk, ad), lambda i: (i, 0)),
                  full((ad, h)), full((1, h))],
        out_specs=row_spec,
        out_shape=jax.ShapeDtypeStruct((n, h), jnp.float32),
    )

    gin = pl.pallas_call(
        _gin_body,
        grid=(grid,),
        in_specs=[row_spec, row_spec, row_spec, full((h, h)), full((1, h))],
        out_specs=row_spec,
        out_shape=jax.ShapeDtypeStruct((n, h), jnp.float32),
    )

    pool = pl.pallas_call(
        functools.partial(_pool_body, nb_graphs=b_graphs),
        in_specs=[pl.BlockSpec((n, h), lambda: (0, 0)),
                  pl.BlockSpec((n, 1), lambda: (0, 0)),
                  pl.BlockSpec((1, h), lambda: (0, 0)),
                  pl.BlockSpec((1, h), lambda: (0, 0)),
                  pl.BlockSpec((1, h), lambda: (0, 0)),
                  pl.BlockSpec((1, 1), lambda: (0, 0)),
                  pl.BlockSpec((h, h), lambda: (0, 0)),
                  pl.BlockSpec((1, h), lambda: (0, 0))],
        out_specs=pl.BlockSpec((b_graphs, h), lambda: (0, 0)),
        out_shape=jax.ShapeDtypeStruct((b_graphs, h), jnp.float32),
    )

    tail = pl.pallas_call(
        _tail_body,
        in_specs=[pl.BlockSpec((b_graphs, h), lambda: (0, 0)),
                  pl.BlockSpec((b_graphs, gd), lambda: (0, 0)),
                  pl.BlockSpec((gd, h), lambda: (0, 0)),
                  pl.BlockSpec((1, h), lambda: (0, 0)),
                  pl.BlockSpec((1, h), lambda: (0, 0)),
                  pl.BlockSpec((1, h), lambda: (0, 0)),
                  pl.BlockSpec((3 * h, 64), lambda: (0, 0)),
                  pl.BlockSpec((1, 64), lambda: (0, 0)),
                  pl.BlockSpec((64, 64), lambda: (0, 0)),
                  pl.BlockSpec((1, 64), lambda: (0, 0)),
                  pl.BlockSpec((1, 64), lambda: (0, 0)),
                  pl.BlockSpec((1, 1), lambda: (0, 0))],
        out_specs=pl.BlockSpec((b_graphs, 1), lambda: (0, 0)),
        out_shape=jax.ShapeDtypeStruct((b_graphs, 1), jnp.float32),
    )

    x0 = atom(graph_x, atom_W, atom_b.reshape(1, h))
    p = segsum(x0, src_r, dst_r)
    x1 = gin(x0, p[0], p[1], gin0_W, gin0_b.reshape(1, h))
    p = segsum(x1, src_r, dst_r)
    x2 = gin(x1, p[0], p[1], gin1_W, gin1_b.reshape(1, h))
    gf = pool(x2, batch_2d, norm_g.reshape(1, h), norm_b.reshape(1, h),
              gate_W.reshape(1, h), gate_b.reshape(1, 1),
              pool_W, pool_b.reshape(1, h))
    return tail(gf, global_x, glob_W, glob_b.reshape(1, h),
                glob_g.reshape(1, h), glob_beta.reshape(1, h),
                emb_W, emb_b.reshape(1, 64), h1_W, h1_b.reshape(1, 64),
                h2_W.reshape(1, 64), h2_b.reshape(1, 1))


# speedup vs baseline: 5.4110x; 1.1609x over previous
import functools

import jax
import jax.numpy as jnp
from jax import lax
from jax.experimental import pallas as pl
from jax.experimental.pallas import tpu as pltpu
from jax.experimental.pallas import tpu_sc as plsc

_INFO = plsc.get_sparse_core_info()
_NC = _INFO.num_cores
_NS = _INFO.num_subcores
_NW = _NC * _NS
_IW = 128
_CH = 64


def _pad_rows(n):
    per_tile = -(-(n + 1) // (_IW * _NS)) * _IW
    return per_tile * _NS, per_tile


def _make_edge_segsum(n, h, nrow):
    npad, rows_per_tile = _pad_rows(n)
    zrep = rows_per_tile // _CH
    mesh = plsc.VectorSubcoreMesh(core_axis_name="c", subcore_axis_name="s")

    @functools.partial(
        pl.kernel,
        mesh=mesh,
        out_type=jax.ShapeDtypeStruct((_NC, npad, h), jnp.float32),
        scratch_types=[
            pltpu.VMEM((nrow, _IW), jnp.int32),
            pltpu.VMEM((nrow, _IW), jnp.int32),
            pltpu.VMEM((_CH,), jnp.int32),
            pltpu.VMEM((_CH,), jnp.int32),
            pltpu.VMEM((_CH, h), jnp.float32),
            pltpu.VMEM((_CH, h), jnp.float32),
            pltpu.SemaphoreType.DMA,
            pltpu.SemaphoreType.DMA,
            pltpu.VMEM_SHARED((npad, h), jnp.float32),
        ],
    )
    def seg(x_hbm, src_hbm, dst_hbm, out_hbm, sidx, didx, d0, d1,
            rows0, rows1, sem0, sem1, acc):
        c = lax.axis_index("c")
        s = lax.axis_index("s")
        wid = s * _NC + c

        pltpu.sync_copy(src_hbm.at[wid], sidx)
        pltpu.sync_copy(dst_hbm.at[wid], didx)

        def _zero(i, carry):
            for l in range(h // 16):
                rows0[i, pl.ds(l * 16, 16)] = jnp.zeros((16,), jnp.float32)
            return carry

        lax.fori_loop(0, _CH, _zero, 0)
        for j in range(zrep):
            pltpu.sync_copy(rows0, acc.at[pl.ds(s * rows_per_tile + j * _CH, _CH)])
        plsc.subcore_barrier()

        def _dcopy(t, dbuf):
            r, o = t // 2, (t % 2) * _CH
            for l in range(_CH // 16):
                dbuf[pl.ds(l * 16, 16)] = didx[r, pl.ds(o + l * 16, 16)]

        def _gather(t, rbuf, sem):
            r, o = t // 2, (t % 2) * _CH
            return pltpu.async_copy(
                x_hbm.at[sidx.at[r, pl.ds(o, _CH)]], rbuf, sem)

        def _gwait(t, rbuf, sem):
            r, o = t // 2, (t % 2) * _CH
            pltpu.make_async_copy(
                x_hbm.at[sidx.at[r, pl.ds(o, _CH)]], rbuf, sem).wait()

        nch = 2 * nrow
        _dcopy(0, d0)
        _gather(0, rows0, sem0)

        def _pair(i, carry):
            g = 2 * i
            _gather(g + 1, rows1, sem1)
            _dcopy(g + 1, d1)
            _gwait(g, rows0, sem0)
            pltpu.sync_copy(rows0, acc.at[d0], add=True)

            @pl.when(g + 2 < nch)
            def _():
                _dcopy(g + 2, d0)
                _gather(g + 2, rows0, sem0)

            _gwait(g + 1, rows1, sem1)
            pltpu.sync_copy(rows1, acc.at[d1], add=True)
            return carry

        lax.fori_loop(0, nrow, _pair, 0)
        plsc.subcore_barrier()
        pltpu.sync_copy(
            acc.at[pl.ds(s * rows_per_tile, rows_per_tile)],
            out_hbm.at[c, pl.ds(s * rows_per_tile, rows_per_tile)],
        )

    return seg


def _atom_body(x_ref, w_ref, b_ref, o_ref):
    o_ref[...] = jnp.maximum(x_ref[...] @ w_ref[...] + b_ref[...], 0.0)


def _gin_body(x_ref, a0_ref, a1_ref, w_ref, b_ref, o_ref):
    x = x_ref[...]
    t = x + a0_ref[...] + a1_ref[...]
    o_ref[...] = x + jnp.maximum(t @ w_ref[...] + b_ref[...], 0.0)


def _pool_body(x_ref, batch_ref, ng_ref, nb_ref, gw_ref, gb_ref, pw_ref,
               pb_ref, gf_ref, *, nb_graphs):
    x = x_ref[...]
    m = jnp.mean(x, axis=1, keepdims=True)
    v = jnp.mean((x - m) * (x - m), axis=1, keepdims=True)
    xln = (x - m) * lax.rsqrt(v + 1e-5) * ng_ref[...] + nb_ref[...]
    gate = jnp.sum(xln * gw_ref[...], axis=1, keepdims=True) + gb_ref[0, 0]
    xn = jnp.maximum(xln @ pw_ref[...] + pb_ref[...], 0.0)
    onehot = (batch_ref[...] ==
              lax.broadcasted_iota(jnp.int32, (1, nb_graphs), 1))
    oh = onehot.astype(jnp.float32)
    gmax = jnp.max(jnp.where(onehot, gate, -1e30), axis=0, keepdims=True)
    gmax_n = jnp.sum(oh * gmax, axis=1, keepdims=True)
    gexp = jnp.exp(gate - gmax_n)
    gsum = jnp.sum(oh * gexp, axis=0, keepdims=True)
    gsum_n = jnp.sum(oh * gsum, axis=1, keepdims=True)
    att = gexp / (gsum_n + 1e-16)
    gf_ref[...] = lax.dot_general(oh * att, xn, (((0,), (0,)), ((), ())),
                                  preferred_element_type=jnp.float32)


def _tail_body(gf_ref, gx_ref, gw_ref, gb_ref, gg_ref, gbeta_ref, ew_ref,
               eb_ref, h1w_ref, h1b_ref, h2w_ref, h2b_ref, o_ref):
    g = gx_ref[...] @ gw_ref[...] + gb_ref[...]
    m = jnp.mean(g, axis=1, keepdims=True)
    v = jnp.mean((g - m) * (g - m), axis=1, keepdims=True)
    gfeat = jnp.maximum(
        (g - m) * lax.rsqrt(v + 1e-5) * gg_ref[...] + gbeta_ref[...], 0.0)
    gf = gf_ref[...]
    merged = jnp.concatenate([gf, gfeat, gf * gfeat], axis=1)
    emb = merged @ ew_ref[...] + eb_ref[...]
    hid = jnp.maximum(emb @ h1w_ref[...] + h1b_ref[...], 0.0)
    o_ref[...] = jnp.sum(hid * h2w_ref[...], axis=1, keepdims=True) + h2b_ref[0, 0]


def kernel(graph_x, global_x, edge_index, batch, atom_W, atom_b, gin0_W,
           gin0_b, gin1_W, gin1_b, norm_g, norm_b, gate_W, gate_b, pool_W,
           pool_b, glob_W, glob_b, glob_g, glob_beta, emb_W, emb_b, h1_W,
           h1_b, h2_W, h2_b):
    n, ad = graph_x.shape
    h = atom_W.shape[1]
    e = edge_index.shape[1]
    b_graphs, gd = global_x.shape
    nrow = -(-e // (_NW * _IW))
    epad = _NW * nrow * _IW - e
    npad, _ = _pad_rows(n)

    src_r = jnp.concatenate(
        [edge_index[0].astype(jnp.int32),
         jnp.zeros((epad,), jnp.int32)]).reshape(_NW, nrow, _IW)
    dst_r = jnp.concatenate(
        [edge_index[1].astype(jnp.int32),
         jnp.full((epad,), npad - 1, jnp.int32)]).reshape(_NW, nrow, _IW)
    batch_2d = batch.astype(jnp.int32).reshape(n, 1)

    segsum = _make_edge_segsum(n, h, nrow)

    blk = 1000
    grid = n // blk
    row_spec = pl.BlockSpec((blk, h), lambda i: (i, 0))
    full = lambda shape: pl.BlockSpec(shape, lambda i: tuple(0 for _ in shape))

    atom = pl.pallas_call(
        _atom_body,
        grid=(grid,),
        in_specs=[pl.BlockSpec((blk, ad), lambda i: (i, 0)),
                  full((ad, h)), full((1, h))],
        out_specs=row_spec,
        out_shape=jax.ShapeDtypeStruct((n, h), jnp.float32),
    )

    gin = pl.pallas_call(
        _gin_body,
        grid=(grid,),
        in_specs=[row_spec, row_spec, row_spec, full((h, h)), full((1, h))],
        out_specs=row_spec,
        out_shape=jax.ShapeDtypeStruct((n, h), jnp.float32),
    )

    pool = pl.pallas_call(
        functools.partial(_pool_body, nb_graphs=b_graphs),
        in_specs=[pl.BlockSpec((n, h), lambda: (0, 0)),
                  pl.BlockSpec((n, 1), lambda: (0, 0)),
                  pl.BlockSpec((1, h), lambda: (0, 0)),
                  pl.BlockSpec((1, h), lambda: (0, 0)),
                  pl.BlockSpec((1, h), lambda: (0, 0)),
                  pl.BlockSpec((1, 1), lambda: (0, 0)),
                  pl.BlockSpec((h, h), lambda: (0, 0)),
                  pl.BlockSpec((1, h), lambda: (0, 0))],
        out_specs=pl.BlockSpec((b_graphs, h), lambda: (0, 0)),
        out_shape=jax.ShapeDtypeStruct((b_graphs, h), jnp.float32),
    )

    tail = pl.pallas_call(
        _tail_body,
        in_specs=[pl.BlockSpec((b_graphs, h), lambda: (0, 0)),
                  pl.BlockSpec((b_graphs, gd), lambda: (0, 0)),
                  pl.BlockSpec((gd, h), lambda: (0, 0)),
                  pl.BlockSpec((1, h), lambda: (0, 0)),
                  pl.BlockSpec((1, h), lambda: (0, 0)),
                  pl.BlockSpec((1, h), lambda: (0, 0)),
                  pl.BlockSpec((3 * h, 64), lambda: (0, 0)),
                  pl.BlockSpec((1, 64), lambda: (0, 0)),
                  pl.BlockSpec((64, 64), lambda: (0, 0)),
                  pl.BlockSpec((1, 64), lambda: (0, 0)),
                  pl.BlockSpec((1, 64), lambda: (0, 0)),
                  pl.BlockSpec((1, 1), lambda: (0, 0))],
        out_specs=pl.BlockSpec((b_graphs, 1), lambda: (0, 0)),
        out_shape=jax.ShapeDtypeStruct((b_graphs, 1), jnp.float32),
    )

    x0 = atom(graph_x, atom_W, atom_b.reshape(1, h))
    p = segsum(x0, src_r, dst_r)
    x1 = gin(x0, p[0], p[1], gin0_W, gin0_b.reshape(1, h))
    p = segsum(x1, src_r, dst_r)
    x2 = gin(x1, p[0], p[1], gin1_W, gin1_b.reshape(1, h))
    gf = pool(x2, batch_2d, norm_g.reshape(1, h), norm_b.reshape(1, h),
              gate_W.reshape(1, h), gate_b.reshape(1, 1),
              pool_W, pool_b.reshape(1, h))
    return tail(gf, global_x, glob_W, glob_b.reshape(1, h),
                glob_g.reshape(1, h), glob_beta.reshape(1, h),
                emb_W, emb_b.reshape(1, 64), h1_W, h1_b.reshape(1, 64),
                h2_W.reshape(1, 64), h2_b.reshape(1, 1))


# speedup vs baseline: 5.4144x; 1.0006x over previous
import functools

import jax
import jax.numpy as jnp
from jax import lax
from jax.experimental import pallas as pl
from jax.experimental.pallas import tpu as pltpu
from jax.experimental.pallas import tpu_sc as plsc

_INFO = plsc.get_sparse_core_info()
_NC = _INFO.num_cores
_NS = _INFO.num_subcores
_NW = _NC * _NS
_IW = 128
_CH = 64


def _pad_rows(n):
    per_tile = -(-(n + 1) // (_IW * _NS)) * _IW
    return per_tile * _NS, per_tile


def _make_edge_segsum(n, h, nrow):
    npad, rows_per_tile = _pad_rows(n)
    zrep = rows_per_tile // _CH
    mesh = plsc.VectorSubcoreMesh(core_axis_name="c", subcore_axis_name="s")

    @functools.partial(
        pl.kernel,
        mesh=mesh,
        out_type=jax.ShapeDtypeStruct((_NC, npad, h), jnp.float32),
        scratch_types=[
            pltpu.VMEM((nrow, _IW), jnp.int32),
            pltpu.VMEM((nrow, _IW), jnp.int32),
            pltpu.VMEM((_CH,), jnp.int32),
            pltpu.VMEM((_CH,), jnp.int32),
            pltpu.VMEM((_CH, h), jnp.float32),
            pltpu.VMEM((_CH, h), jnp.float32),
            pltpu.SemaphoreType.DMA,
            pltpu.SemaphoreType.DMA,
            pltpu.VMEM_SHARED((npad, h), jnp.float32),
        ],
    )
    def seg(x_hbm, src_hbm, dst_hbm, out_hbm, sidx, didx, d0, d1,
            rows0, rows1, sem0, sem1, acc):
        c = lax.axis_index("c")
        s = lax.axis_index("s")
        wid = s * _NC + c

        pltpu.sync_copy(src_hbm.at[wid], sidx)
        pltpu.sync_copy(dst_hbm.at[wid], didx)

        def _zero(i, carry):
            for l in range(h // 16):
                rows0[i, pl.ds(l * 16, 16)] = jnp.zeros((16,), jnp.float32)
            return carry

        lax.fori_loop(0, _CH, _zero, 0)
        for j in range(zrep):
            pltpu.sync_copy(rows0, acc.at[pl.ds(s * rows_per_tile + j * _CH, _CH)])
        plsc.subcore_barrier()

        def _dcopy(t, dbuf):
            r, o = t // 2, (t % 2) * _CH
            for l in range(_CH // 16):
                dbuf[pl.ds(l * 16, 16)] = didx[r, pl.ds(o + l * 16, 16)]

        def _gather(t, rbuf, sem):
            r, o = t // 2, (t % 2) * _CH
            return pltpu.async_copy(
                x_hbm.at[sidx.at[r, pl.ds(o, _CH)]], rbuf, sem)

        def _gwait(t, rbuf, sem):
            r, o = t // 2, (t % 2) * _CH
            pltpu.make_async_copy(
                x_hbm.at[sidx.at[r, pl.ds(o, _CH)]], rbuf, sem).wait()

        nch = 2 * nrow
        _dcopy(0, d0)
        _gather(0, rows0, sem0)

        def _pair(i, carry):
            g = 2 * i
            _gather(g + 1, rows1, sem1)
            _dcopy(g + 1, d1)
            _gwait(g, rows0, sem0)
            pltpu.sync_copy(rows0, acc.at[d0], add=True)

            @pl.when(g + 2 < nch)
            def _():
                _dcopy(g + 2, d0)
                _gather(g + 2, rows0, sem0)

            _gwait(g + 1, rows1, sem1)
            pltpu.sync_copy(rows1, acc.at[d1], add=True)
            return carry

        lax.fori_loop(0, nrow, _pair, 0)
        plsc.subcore_barrier()
        pltpu.sync_copy(
            acc.at[pl.ds(s * rows_per_tile, rows_per_tile)],
            out_hbm.at[c, pl.ds(s * rows_per_tile, rows_per_tile)],
        )

    return seg


def _atom_body(x_ref, w_ref, b_ref, o_ref):
    o_ref[...] = jnp.maximum(x_ref[...] @ w_ref[...] + b_ref[...], 0.0)


def _gin_body(x_ref, a0_ref, a1_ref, w_ref, b_ref, o_ref):
    x = x_ref[...]
    t = x + a0_ref[...] + a1_ref[...]
    o_ref[...] = x + jnp.maximum(t @ w_ref[...] + b_ref[...], 0.0)


def _pool_body(x_ref, batch_ref, ng_ref, nb_ref, gw_ref, gb_ref, pw_ref,
               pb_ref, gf_ref, *, nb_graphs):
    x = x_ref[...]
    m = jnp.mean(x, axis=1, keepdims=True)
    v = jnp.mean((x - m) * (x - m), axis=1, keepdims=True)
    xln = (x - m) * lax.rsqrt(v + 1e-5) * ng_ref[...] + nb_ref[...]
    gate = jnp.sum(xln * gw_ref[...], axis=1, keepdims=True) + gb_ref[0, 0]
    xn = jnp.maximum(xln @ pw_ref[...] + pb_ref[...], 0.0)
    onehot = (batch_ref[...] ==
              lax.broadcasted_iota(jnp.int32, (1, nb_graphs), 1))
    oh = onehot.astype(jnp.float32)
    gmax = jnp.max(jnp.where(onehot, gate, -1e30), axis=0, keepdims=True)
    gmax_n = jnp.sum(oh * gmax, axis=1, keepdims=True)
    gexp = jnp.exp(gate - gmax_n)
    gsum = jnp.sum(oh * gexp, axis=0, keepdims=True)
    gsum_n = jnp.sum(oh * gsum, axis=1, keepdims=True)
    att = gexp / (gsum_n + 1e-16)
    gf_ref[...] = lax.dot_general(oh * att, xn, (((0,), (0,)), ((), ())),
                                  preferred_element_type=jnp.float32)


def _tail_body(gf_ref, gx_ref, gw_ref, gb_ref, gg_ref, gbeta_ref, ew_ref,
               eb_ref, h1w_ref, h1b_ref, h2w_ref, h2b_ref, o_ref):
    g = gx_ref[...] @ gw_ref[...] + gb_ref[...]
    m = jnp.mean(g, axis=1, keepdims=True)
    v = jnp.mean((g - m) * (g - m), axis=1, keepdims=True)
    gfeat = jnp.maximum(
        (g - m) * lax.rsqrt(v + 1e-5) * gg_ref[...] + gbeta_ref[...], 0.0)
    gf = gf_ref[...]
    merged = jnp.concatenate([gf, gfeat, gf * gfeat], axis=1)
    emb = merged @ ew_ref[...] + eb_ref[...]
    hid = jnp.maximum(emb @ h1w_ref[...] + h1b_ref[...], 0.0)
    o_ref[...] = jnp.sum(hid * h2w_ref[...], axis=1, keepdims=True) + h2b_ref[0, 0]


def kernel(graph_x, global_x, edge_index, batch, atom_W, atom_b, gin0_W,
           gin0_b, gin1_W, gin1_b, norm_g, norm_b, gate_W, gate_b, pool_W,
           pool_b, glob_W, glob_b, glob_g, glob_beta, emb_W, emb_b, h1_W,
           h1_b, h2_W, h2_b):
    n, ad = graph_x.shape
    h = atom_W.shape[1]
    e = edge_index.shape[1]
    b_graphs, gd = global_x.shape
    nrow = -(-e // (_NW * _IW))
    epad = _NW * nrow * _IW - e
    npad, _ = _pad_rows(n)

    src_r = jnp.concatenate(
        [edge_index[0].astype(jnp.int32),
         jnp.zeros((epad,), jnp.int32)]).reshape(_NW, nrow, _IW)
    dst_r = jnp.concatenate(
        [edge_index[1].astype(jnp.int32),
         n + jnp.arange(epad, dtype=jnp.int32) % (npad - n)]
    ).reshape(_NW, nrow, _IW)
    batch_2d = batch.astype(jnp.int32).reshape(n, 1)

    segsum = _make_edge_segsum(n, h, nrow)

    blk = 1000
    grid = n // blk
    row_spec = pl.BlockSpec((blk, h), lambda i: (i, 0))
    full = lambda shape: pl.BlockSpec(shape, lambda i: tuple(0 for _ in shape))

    atom = pl.pallas_call(
        _atom_body,
        grid=(grid,),
        in_specs=[pl.BlockSpec((blk, ad), lambda i: (i, 0)),
                  full((ad, h)), full((1, h))],
        out_specs=row_spec,
        out_shape=jax.ShapeDtypeStruct((n, h), jnp.float32),
    )

    gin = pl.pallas_call(
        _gin_body,
        grid=(grid,),
        in_specs=[row_spec, row_spec, row_spec, full((h, h)), full((1, h))],
        out_specs=row_spec,
        out_shape=jax.ShapeDtypeStruct((n, h), jnp.float32),
    )

    pool = pl.pallas_call(
        functools.partial(_pool_body, nb_graphs=b_graphs),
        in_specs=[pl.BlockSpec((n, h), lambda: (0, 0)),
                  pl.BlockSpec((n, 1), lambda: (0, 0)),
                  pl.BlockSpec((1, h), lambda: (0, 0)),
                  pl.BlockSpec((1, h), lambda: (0, 0)),
                  pl.BlockSpec((1, h), lambda: (0, 0)),
                  pl.BlockSpec((1, 1), lambda: (0, 0)),
                  pl.BlockSpec((h, h), lambda: (0, 0)),
                  pl.BlockSpec((1, h), lambda: (0, 0))],
        out_specs=pl.BlockSpec((b_graphs, h), lambda: (0, 0)),
        out_shape=jax.ShapeDtypeStruct((b_graphs, h), jnp.float32),
    )

    tail = pl.pallas_call(
        _tail_body,
        in_specs=[pl.BlockSpec((b_graphs, h), lambda: (0, 0)),
                  pl.BlockSpec((b_graphs, gd), lambda: (0, 0)),
                  pl.BlockSpec((gd, h), lambda: (0, 0)),
                  pl.BlockSpec((1, h), lambda: (0, 0)),
                  pl.BlockSpec((1, h), lambda: (0, 0)),
                  pl.BlockSpec((1, h), lambda: (0, 0)),
                  pl.BlockSpec((3 * h, 64), lambda: (0, 0)),
                  pl.BlockSpec((1, 64), lambda: (0, 0)),
                  pl.BlockSpec((64, 64), lambda: (0, 0)),
                  pl.BlockSpec((1, 64), lambda: (0, 0)),
                  pl.BlockSpec((1, 64), lambda: (0, 0)),
                  pl.BlockSpec((1, 1), lambda: (0, 0))],
        out_specs=pl.BlockSpec((b_graphs, 1), lambda: (0, 0)),
        out_shape=jax.ShapeDtypeStruct((b_graphs, 1), jnp.float32),
    )

    x0 = atom(graph_x, atom_W, atom_b.reshape(1, h))
    p = segsum(x0, src_r, dst_r)
    x1 = gin(x0, p[0], p[1], gin0_W, gin0_b.reshape(1, h))
    p = segsum(x1, src_r, dst_r)
    x2 = gin(x1, p[0], p[1], gin1_W, gin1_b.reshape(1, h))
    gf = pool(x2, batch_2d, norm_g.reshape(1, h), norm_b.reshape(1, h),
              gate_W.reshape(1, h), gate_b.reshape(1, 1),
              pool_W, pool_b.reshape(1, h))
    return tail(gf, global_x, glob_W, glob_b.reshape(1, h),
                glob_g.reshape(1, h), glob_beta.reshape(1, h),
                emb_W, emb_b.reshape(1, 64), h1_W, h1_b.reshape(1, 64),
                h2_W.reshape(1, 64), h2_b.reshape(1, 1))


# speedup vs baseline: 9.5682x; 1.7672x over previous
import functools

import jax
import jax.numpy as jnp
from jax import lax
from jax.experimental import pallas as pl
from jax.experimental.pallas import tpu as pltpu
from jax.experimental.pallas import tpu_sc as plsc

_INFO = plsc.get_sparse_core_info()
_NC = _INFO.num_cores
_NS = _INFO.num_subcores
_NW = _NC * _NS
_IW = 128
_CH = 64


def _pad_rows(n):
    per_tile = -(-(n + 1) // (_IW * _NS)) * _IW
    return per_tile * _NS, per_tile


def _make_edge_segsum(n, h, nrow):
    npad, rows_per_tile = _pad_rows(n)
    zrep = rows_per_tile // _CH
    mesh = plsc.VectorSubcoreMesh(core_axis_name="c", subcore_axis_name="s")

    @functools.partial(
        pl.kernel,
        mesh=mesh,
        out_type=jax.ShapeDtypeStruct((_NC, npad, h), jnp.float32),
        scratch_types=[
            pltpu.VMEM((nrow, _IW), jnp.int32),
            pltpu.VMEM((nrow, _IW), jnp.int32),
            pltpu.VMEM((_CH,), jnp.int32),
            pltpu.VMEM((_CH,), jnp.int32),
            pltpu.VMEM((_CH, h), jnp.float32),
            pltpu.VMEM((_CH, h), jnp.float32),
            pltpu.SemaphoreType.DMA,
            pltpu.SemaphoreType.DMA,
            pltpu.VMEM_SHARED((npad, h), jnp.float32),
        ],
    )
    def seg(x_hbm, src_hbm, dst_hbm, out_hbm, sidx, didx, d0, d1,
            rows0, rows1, sem0, sem1, acc):
        c = lax.axis_index("c")
        s = lax.axis_index("s")
        wid = s * _NC + c

        pltpu.sync_copy(src_hbm.at[wid], sidx)
        pltpu.sync_copy(dst_hbm.at[wid], didx)

        def _zero(i, carry):
            for l in range(h // 16):
                rows0[i, pl.ds(l * 16, 16)] = jnp.zeros((16,), jnp.float32)
            return carry

        lax.fori_loop(0, _CH, _zero, 0)
        for j in range(zrep):
            pltpu.sync_copy(rows0, acc.at[pl.ds(s * rows_per_tile + j * _CH, _CH)])
        plsc.subcore_barrier()

        def _dcopy(t, dbuf):
            r, o = t // 2, (t % 2) * _CH
            for l in range(_CH // 16):
                dbuf[pl.ds(l * 16, 16)] = didx[r, pl.ds(o + l * 16, 16)]

        def _gather(t, rbuf, sem):
            r, o = t // 2, (t % 2) * _CH
            return pltpu.async_copy(
                x_hbm.at[sidx.at[r, pl.ds(o, _CH)]], rbuf, sem)

        def _gwait(t, rbuf, sem):
            r, o = t // 2, (t % 2) * _CH
            pltpu.make_async_copy(
                x_hbm.at[sidx.at[r, pl.ds(o, _CH)]], rbuf, sem).wait()

        nch = 2 * nrow
        _dcopy(0, d0)
        _gather(0, rows0, sem0)

        def _pair(i, carry):
            g = 2 * i
            _gather(g + 1, rows1, sem1)
            _dcopy(g + 1, d1)
            _gwait(g, rows0, sem0)
            pltpu.sync_copy(rows0, acc.at[d0], add=True)

            @pl.when(g + 2 < nch)
            def _():
                _dcopy(g + 2, d0)
                _gather(g + 2, rows0, sem0)

            _gwait(g + 1, rows1, sem1)
            pltpu.sync_copy(rows1, acc.at[d1], add=True)
            return carry

        lax.fori_loop(0, nrow, _pair, 0)
        plsc.subcore_barrier()
        pltpu.sync_copy(
            acc.at[pl.ds(s * rows_per_tile, rows_per_tile)],
            out_hbm.at[c, pl.ds(s * rows_per_tile, rows_per_tile)],
        )

    return seg


def _atom_body(x_ref, w_ref, b_ref, o_ref):
    o_ref[...] = jnp.maximum(x_ref[...] @ w_ref[...] + b_ref[...], 0.0)


def _gin_body(x_ref, a0_ref, a1_ref, w_ref, b_ref, o_ref):
    x = x_ref[...]
    t = x + a0_ref[...] + a1_ref[...]
    o_ref[...] = x + jnp.maximum(t @ w_ref[...] + b_ref[...], 0.0)


def _pool_body(x_ref, batch_ref, ng_ref, nb_ref, gw_ref, gb_ref, pw_ref,
               pb_ref, gf_ref, *, nb_graphs):
    x = x_ref[...]
    m = jnp.mean(x, axis=1, keepdims=True)
    v = jnp.mean((x - m) * (x - m), axis=1, keepdims=True)
    xln = (x - m) * lax.rsqrt(v + 1e-5) * ng_ref[...] + nb_ref[...]
    gate = jnp.sum(xln * gw_ref[...], axis=1, keepdims=True) + gb_ref[0, 0]
    xn = jnp.maximum(xln @ pw_ref[...] + pb_ref[...], 0.0)
    onehot = (batch_ref[...] ==
              lax.broadcasted_iota(jnp.int32, (1, nb_graphs), 1))
    oh = onehot.astype(jnp.float32)
    gmax = jnp.max(jnp.where(onehot, gate, -1e30), axis=0, keepdims=True)
    gmax_n = jnp.sum(oh * gmax, axis=1, keepdims=True)
    gexp = jnp.exp(gate - gmax_n)
    gsum = jnp.sum(oh * gexp, axis=0, keepdims=True)
    gsum_n = jnp.sum(oh * gsum, axis=1, keepdims=True)
    att = gexp / (gsum_n + 1e-16)
    gf_ref[...] = lax.dot_general(oh * att, xn, (((0,), (0,)), ((), ())),
                                  preferred_element_type=jnp.float32)


def _tail_body(gf_ref, gx_ref, gw_ref, gb_ref, gg_ref, gbeta_ref, ew_ref,
               eb_ref, h1w_ref, h1b_ref, h2w_ref, h2b_ref, o_ref):
    g = gx_ref[...] @ gw_ref[...] + gb_ref[...]
    m = jnp.mean(g, axis=1, keepdims=True)
    v = jnp.mean((g - m) * (g - m), axis=1, keepdims=True)
    gfeat = jnp.maximum(
        (g - m) * lax.rsqrt(v + 1e-5) * gg_ref[...] + gbeta_ref[...], 0.0)
    gf = gf_ref[...]
    merged = jnp.concatenate([gf, gfeat, gf * gfeat], axis=1)
    emb = merged @ ew_ref[...] + eb_ref[...]
    hid = jnp.maximum(emb @ h1w_ref[...] + h1b_ref[...], 0.0)
    o_ref[...] = jnp.sum(hid * h2w_ref[...], axis=1, keepdims=True) + h2b_ref[0, 0]


def kernel(graph_x, global_x, edge_index, batch, atom_W, atom_b, gin0_W,
           gin0_b, gin1_W, gin1_b, norm_g, norm_b, gate_W, gate_b, pool_W,
           pool_b, glob_W, glob_b, glob_g, glob_beta, emb_W, emb_b, h1_W,
           h1_b, h2_W, h2_b):
    n, ad = graph_x.shape
    h = atom_W.shape[1]
    e = edge_index.shape[1]
    b_graphs, gd = global_x.shape
    nrow = -(-e // (_NW * _IW))
    epad = _NW * nrow * _IW - e
    npad, _ = _pad_rows(n)

    del epad
    src_all = edge_index[0].astype(jnp.int32)
    dst_all = edge_index[1].astype(jnp.int32)
    slots = nrow * _IW
    base, rem = divmod(e, _NW)
    src_parts, dst_parts = [], []
    off = 0
    for w in range(_NW):
        cw = base + (1 if w < rem else 0)
        pad = slots - cw
        pr = jnp.arange(pad, dtype=jnp.int32)
        src_parts.append(jnp.concatenate([src_all[off:off + cw], pr % n]))
        dst_parts.append(jnp.concatenate(
            [dst_all[off:off + cw], n + (w * 7 + pr) % (npad - n)]))
        off += cw
    src_r = jnp.stack(src_parts).reshape(_NW, nrow, _IW)
    dst_r = jnp.stack(dst_parts).reshape(_NW, nrow, _IW)
    batch_2d = batch.astype(jnp.int32).reshape(n, 1)

    segsum = _make_edge_segsum(n, h, nrow)

    blk = 1000
    grid = n // blk
    row_spec = pl.BlockSpec((blk, h), lambda i: (i, 0))
    full = lambda shape: pl.BlockSpec(shape, lambda i: tuple(0 for _ in shape))

    atom = pl.pallas_call(
        _atom_body,
        grid=(grid,),
        in_specs=[pl.BlockSpec((blk, ad), lambda i: (i, 0)),
                  full((ad, h)), full((1, h))],
        out_specs=row_spec,
        out_shape=jax.ShapeDtypeStruct((n, h), jnp.float32),
    )

    gin = pl.pallas_call(
        _gin_body,
        grid=(grid,),
        in_specs=[row_spec, row_spec, row_spec, full((h, h)), full((1, h))],
        out_specs=row_spec,
        out_shape=jax.ShapeDtypeStruct((n, h), jnp.float32),
    )

    pool = pl.pallas_call(
        functools.partial(_pool_body, nb_graphs=b_graphs),
        in_specs=[pl.BlockSpec((n, h), lambda: (0, 0)),
                  pl.BlockSpec((n, 1), lambda: (0, 0)),
                  pl.BlockSpec((1, h), lambda: (0, 0)),
                  pl.BlockSpec((1, h), lambda: (0, 0)),
                  pl.BlockSpec((1, h), lambda: (0, 0)),
                  pl.BlockSpec((1, 1), lambda: (0, 0)),
                  pl.BlockSpec((h, h), lambda: (0, 0)),
                  pl.BlockSpec((1, h), lambda: (0, 0))],
        out_specs=pl.BlockSpec((b_graphs, h), lambda: (0, 0)),
        out_shape=jax.ShapeDtypeStruct((b_graphs, h), jnp.float32),
    )

    tail = pl.pallas_call(
        _tail_body,
        in_specs=[pl.BlockSpec((b_graphs, h), lambda: (0, 0)),
                  pl.BlockSpec((b_graphs, gd), lambda: (0, 0)),
                  pl.BlockSpec((gd, h), lambda: (0, 0)),
                  pl.BlockSpec((1, h), lambda: (0, 0)),
                  pl.BlockSpec((1, h), lambda: (0, 0)),
                  pl.BlockSpec((1, h), lambda: (0, 0)),
                  pl.BlockSpec((3 * h, 64), lambda: (0, 0)),
                  pl.BlockSpec((1, 64), lambda: (0, 0)),
                  pl.BlockSpec((64, 64), lambda: (0, 0)),
                  pl.BlockSpec((1, 64), lambda: (0, 0)),
                  pl.BlockSpec((1, 64), lambda: (0, 0)),
                  pl.BlockSpec((1, 1), lambda: (0, 0))],
        out_specs=pl.BlockSpec((b_graphs, 1), lambda: (0, 0)),
        out_shape=jax.ShapeDtypeStruct((b_graphs, 1), jnp.float32),
    )

    x0 = atom(graph_x, atom_W, atom_b.reshape(1, h))
    p = segsum(x0, src_r, dst_r)
    x1 = gin(x0, p[0], p[1], gin0_W, gin0_b.reshape(1, h))
    p = segsum(x1, src_r, dst_r)
    x2 = gin(x1, p[0], p[1], gin1_W, gin1_b.reshape(1, h))
    gf = pool(x2, batch_2d, norm_g.reshape(1, h), norm_b.reshape(1, h),
              gate_W.reshape(1, h), gate_b.reshape(1, 1),
              pool_W, pool_b.reshape(1, h))
    return tail(gf, global_x, glob_W, glob_b.reshape(1, h),
                glob_g.reshape(1, h), glob_beta.reshape(1, h),
                emb_W, emb_b.reshape(1, 64), h1_W, h1_b.reshape(1, 64),
                h2_W.reshape(1, 64), h2_b.reshape(1, 1))


# speedup vs baseline: 9.9837x; 1.0434x over previous
import functools

import jax
import jax.numpy as jnp
from jax import lax
from jax.experimental import pallas as pl
from jax.experimental.pallas import tpu as pltpu
from jax.experimental.pallas import tpu_sc as plsc

_INFO = plsc.get_sparse_core_info()
_NC = _INFO.num_cores
_NS = _INFO.num_subcores
_NW = _NC * _NS
_IW = 128
_CH = 64


def _pad_rows(n):
    per_tile = -(-(n + 1) // (_IW * _NS)) * _IW
    return per_tile * _NS, per_tile


def _make_edge_segsum(n, h, nrow):
    npad, rows_per_tile = _pad_rows(n)
    zrep = rows_per_tile // _CH
    mesh = plsc.VectorSubcoreMesh(core_axis_name="c", subcore_axis_name="s")

    @functools.partial(
        pl.kernel,
        mesh=mesh,
        out_type=jax.ShapeDtypeStruct((_NC, npad, h), jnp.float32),
        scratch_types=[
            pltpu.VMEM((nrow, _IW), jnp.int32),
            pltpu.VMEM((nrow, _IW), jnp.int32),
            pltpu.VMEM((_CH,), jnp.int32),
            pltpu.VMEM((_CH,), jnp.int32),
            pltpu.VMEM((_CH, h), jnp.float32),
            pltpu.VMEM((_CH, h), jnp.float32),
            pltpu.SemaphoreType.DMA,
            pltpu.SemaphoreType.DMA,
            pltpu.VMEM_SHARED((npad, h), jnp.float32),
        ],
    )
    def seg(x_hbm, src_hbm, dst_hbm, out_hbm, sidx, didx, d0, d1,
            rows0, rows1, sem0, sem1, acc):
        c = lax.axis_index("c")
        s = lax.axis_index("s")
        wid = s * _NC + c

        pltpu.sync_copy(src_hbm.at[wid], sidx)
        pltpu.sync_copy(dst_hbm.at[wid], didx)

        def _zero(i, carry):
            for l in range(h // 16):
                rows0[i, pl.ds(l * 16, 16)] = jnp.zeros((16,), jnp.float32)
            return carry

        lax.fori_loop(0, _CH, _zero, 0)
        for j in range(zrep):
            pltpu.sync_copy(rows0, acc.at[pl.ds(s * rows_per_tile + j * _CH, _CH)])
        plsc.subcore_barrier()

        def _dcopy(t, dbuf):
            r, o = t // 2, (t % 2) * _CH
            for l in range(_CH // 16):
                dbuf[pl.ds(l * 16, 16)] = didx[r, pl.ds(o + l * 16, 16)]

        def _gather(t, rbuf, sem):
            r, o = t // 2, (t % 2) * _CH
            return pltpu.async_copy(
                x_hbm.at[sidx.at[r, pl.ds(o, _CH)]], rbuf, sem)

        def _gwait(t, rbuf, sem):
            r, o = t // 2, (t % 2) * _CH
            pltpu.make_async_copy(
                x_hbm.at[sidx.at[r, pl.ds(o, _CH)]], rbuf, sem).wait()

        nch = 2 * nrow
        _dcopy(0, d0)
        _gather(0, rows0, sem0)

        def _pair(i, carry):
            g = 2 * i
            _gather(g + 1, rows1, sem1)
            _dcopy(g + 1, d1)
            _gwait(g, rows0, sem0)
            pltpu.sync_copy(rows0, acc.at[d0], add=True)

            @pl.when(g + 2 < nch)
            def _():
                _dcopy(g + 2, d0)
                _gather(g + 2, rows0, sem0)

            _gwait(g + 1, rows1, sem1)
            pltpu.sync_copy(rows1, acc.at[d1], add=True)
            return carry

        lax.fori_loop(0, nrow, _pair, 0)
        plsc.subcore_barrier()
        pltpu.sync_copy(
            acc.at[pl.ds(s * rows_per_tile, rows_per_tile)],
            out_hbm.at[c, pl.ds(s * rows_per_tile, rows_per_tile)],
        )

    return seg


def _atom_body(x_ref, w_ref, b_ref, o_ref):
    o_ref[...] = jnp.maximum(x_ref[...] @ w_ref[...] + b_ref[...], 0.0)


def _gin_body(x_ref, a0_ref, a1_ref, w_ref, b_ref, o_ref):
    x = x_ref[...]
    t = x + a0_ref[0] + a1_ref[0]
    o_ref[...] = x + jnp.maximum(t @ w_ref[...] + b_ref[...], 0.0)


def _pool_body(x_ref, batch_ref, ng_ref, nb_ref, gw_ref, gb_ref, pw_ref,
               pb_ref, gf_ref, *, nb_graphs):
    x = x_ref[...]
    m = jnp.mean(x, axis=1, keepdims=True)
    v = jnp.mean((x - m) * (x - m), axis=1, keepdims=True)
    xln = (x - m) * lax.rsqrt(v + 1e-5) * ng_ref[...] + nb_ref[...]
    gate = jnp.sum(xln * gw_ref[...], axis=1, keepdims=True) + gb_ref[0, 0]
    xn = jnp.maximum(xln @ pw_ref[...] + pb_ref[...], 0.0)
    onehot = (batch_ref[...] ==
              lax.broadcasted_iota(jnp.int32, (1, nb_graphs), 1))
    oh = onehot.astype(jnp.float32)
    gmax = jnp.max(jnp.where(onehot, gate, -1e30), axis=0, keepdims=True)
    gmax_n = jnp.sum(oh * gmax, axis=1, keepdims=True)
    gexp = jnp.exp(gate - gmax_n)
    gsum = jnp.sum(oh * gexp, axis=0, keepdims=True)
    gsum_n = jnp.sum(oh * gsum, axis=1, keepdims=True)
    att = gexp / (gsum_n + 1e-16)
    gf_ref[...] = lax.dot_general(oh * att, xn, (((0,), (0,)), ((), ())),
                                  preferred_element_type=jnp.float32)


def _tail_body(gf_ref, gx_ref, gw_ref, gb_ref, gg_ref, gbeta_ref, ew_ref,
               eb_ref, h1w_ref, h1b_ref, h2w_ref, h2b_ref, o_ref):
    g = gx_ref[...] @ gw_ref[...] + gb_ref[...]
    m = jnp.mean(g, axis=1, keepdims=True)
    v = jnp.mean((g - m) * (g - m), axis=1, keepdims=True)
    gfeat = jnp.maximum(
        (g - m) * lax.rsqrt(v + 1e-5) * gg_ref[...] + gbeta_ref[...], 0.0)
    gf = gf_ref[...]
    merged = jnp.concatenate([gf, gfeat, gf * gfeat], axis=1)
    emb = merged @ ew_ref[...] + eb_ref[...]
    hid = jnp.maximum(emb @ h1w_ref[...] + h1b_ref[...], 0.0)
    o_ref[...] = jnp.sum(hid * h2w_ref[...], axis=1, keepdims=True) + h2b_ref[0, 0]


def kernel(graph_x, global_x, edge_index, batch, atom_W, atom_b, gin0_W,
           gin0_b, gin1_W, gin1_b, norm_g, norm_b, gate_W, gate_b, pool_W,
           pool_b, glob_W, glob_b, glob_g, glob_beta, emb_W, emb_b, h1_W,
           h1_b, h2_W, h2_b):
    n, ad = graph_x.shape
    h = atom_W.shape[1]
    e = edge_index.shape[1]
    b_graphs, gd = global_x.shape
    nrow = -(-e // (_NW * _IW))
    epad = _NW * nrow * _IW - e
    npad, _ = _pad_rows(n)

    del epad
    src_all = edge_index[0].astype(jnp.int32)
    dst_all = edge_index[1].astype(jnp.int32)
    slots = nrow * _IW
    base, rem = divmod(e, _NW)
    src_parts, dst_parts = [], []
    off = 0
    for w in range(_NW):
        cw = base + (1 if w < rem else 0)
        pad = slots - cw
        pr = jnp.arange(pad, dtype=jnp.int32)
        src_parts.append(jnp.concatenate([src_all[off:off + cw], pr % n]))
        dst_parts.append(jnp.concatenate(
            [dst_all[off:off + cw], n + (w * 7 + pr) % (npad - n)]))
        off += cw
    src_r = jnp.stack(src_parts).reshape(_NW, nrow, _IW)
    dst_r = jnp.stack(dst_parts).reshape(_NW, nrow, _IW)
    batch_2d = batch.astype(jnp.int32).reshape(n, 1)

    segsum = _make_edge_segsum(n, h, nrow)

    blk = 1000
    grid = n // blk
    row_spec = pl.BlockSpec((blk, h), lambda i: (i, 0))
    full = lambda shape: pl.BlockSpec(shape, lambda i: tuple(0 for _ in shape))

    atom = pl.pallas_call(
        _atom_body,
        grid=(grid,),
        in_specs=[pl.BlockSpec((blk, ad), lambda i: (i, 0)),
                  full((ad, h)), full((1, h))],
        out_specs=row_spec,
        out_shape=jax.ShapeDtypeStruct((n, h), jnp.float32),
    )

    part0 = pl.BlockSpec((1, blk, h), lambda i: (0, i, 0))
    part1 = pl.BlockSpec((1, blk, h), lambda i: (1, i, 0))
    gin = pl.pallas_call(
        _gin_body,
        grid=(grid,),
        in_specs=[row_spec, part0, part1, full((h, h)), full((1, h))],
        out_specs=row_spec,
        out_shape=jax.ShapeDtypeStruct((n, h), jnp.float32),
    )

    pool = pl.pallas_call(
        functools.partial(_pool_body, nb_graphs=b_graphs),
        in_specs=[pl.BlockSpec((n, h), lambda: (0, 0)),
                  pl.BlockSpec((n, 1), lambda: (0, 0)),
                  pl.BlockSpec((1, h), lambda: (0, 0)),
                  pl.BlockSpec((1, h), lambda: (0, 0)),
                  pl.BlockSpec((1, h), lambda: (0, 0)),
                  pl.BlockSpec((1, 1), lambda: (0, 0)),
                  pl.BlockSpec((h, h), lambda: (0, 0)),
                  pl.BlockSpec((1, h), lambda: (0, 0))],
        out_specs=pl.BlockSpec((b_graphs, h), lambda: (0, 0)),
        out_shape=jax.ShapeDtypeStruct((b_graphs, h), jnp.float32),
    )

    tail = pl.pallas_call(
        _tail_body,
        in_specs=[pl.BlockSpec((b_graphs, h), lambda: (0, 0)),
                  pl.BlockSpec((b_graphs, gd), lambda: (0, 0)),
                  pl.BlockSpec((gd, h), lambda: (0, 0)),
                  pl.BlockSpec((1, h), lambda: (0, 0)),
                  pl.BlockSpec((1, h), lambda: (0, 0)),
                  pl.BlockSpec((1, h), lambda: (0, 0)),
                  pl.BlockSpec((3 * h, 64), lambda: (0, 0)),
                  pl.BlockSpec((1, 64), lambda: (0, 0)),
                  pl.BlockSpec((64, 64), lambda: (0, 0)),
                  pl.BlockSpec((1, 64), lambda: (0, 0)),
                  pl.BlockSpec((1, 64), lambda: (0, 0)),
                  pl.BlockSpec((1, 1), lambda: (0, 0))],
        out_specs=pl.BlockSpec((b_graphs, 1), lambda: (0, 0)),
        out_shape=jax.ShapeDtypeStruct((b_graphs, 1), jnp.float32),
    )

    x0 = atom(graph_x, atom_W, atom_b.reshape(1, h))
    p = segsum(x0, src_r, dst_r)
    x1 = gin(x0, p, p, gin0_W, gin0_b.reshape(1, h))
    p = segsum(x1, src_r, dst_r)
    x2 = gin(x1, p, p, gin1_W, gin1_b.reshape(1, h))
    gf = pool(x2, batch_2d, norm_g.reshape(1, h), norm_b.reshape(1, h),
              gate_W.reshape(1, h), gate_b.reshape(1, 1),
              pool_W, pool_b.reshape(1, h))
    return tail(gf, global_x, glob_W, glob_b.reshape(1, h),
                glob_g.reshape(1, h), glob_beta.reshape(1, h),
                emb_W, emb_b.reshape(1, 64), h1_W, h1_b.reshape(1, 64),
                h2_W.reshape(1, 64), h2_b.reshape(1, 1))


# speedup vs baseline: 11.5014x; 1.1520x over previous
import functools

import jax
import jax.numpy as jnp
from jax import lax
from jax.experimental import pallas as pl
from jax.experimental.pallas import tpu as pltpu
from jax.experimental.pallas import tpu_sc as plsc

_INFO = plsc.get_sparse_core_info()
_NC = _INFO.num_cores
_NS = _INFO.num_subcores
_NW = _NC * _NS
_IW = 128
_CH = 128
_RPC = _IW // _CH


def _pad_rows(n):
    per_tile = -(-(n + 1) // (_IW * _NS)) * _IW
    return per_tile * _NS, per_tile


def _make_edge_segsum(n, h, nrow):
    npad, rows_per_tile = _pad_rows(n)
    zrep = rows_per_tile // _CH
    mesh = plsc.VectorSubcoreMesh(core_axis_name="c", subcore_axis_name="s")

    @functools.partial(
        pl.kernel,
        mesh=mesh,
        out_type=jax.ShapeDtypeStruct((_NC, npad, h), jnp.float32),
        scratch_types=[
            pltpu.VMEM((nrow, _IW), jnp.int32),
            pltpu.VMEM((_CH,), jnp.int32),
            pltpu.VMEM((_CH,), jnp.int32),
            pltpu.VMEM((_CH, h), jnp.float32),
            pltpu.VMEM((_CH, h), jnp.float32),
            pltpu.SemaphoreType.DMA,
            pltpu.SemaphoreType.DMA,
            pltpu.SemaphoreType.DMA,
            pltpu.SemaphoreType.DMA,
            pltpu.VMEM_SHARED((npad, h), jnp.float32),
        ],
    )
    def seg(x_hbm, src_hbm, dst_hbm, out_hbm, sidx, d0, d1,
            rows0, rows1, sem0, sem1, semd0, semd1, acc):
        c = lax.axis_index("c")
        s = lax.axis_index("s")
        wid = s * _NC + c

        pltpu.sync_copy(src_hbm.at[wid], sidx)

        def _zero(i, carry):
            for l in range(h // 16):
                rows0[i, pl.ds(l * 16, 16)] = jnp.zeros((16,), jnp.float32)
            return carry

        lax.fori_loop(0, _CH, _zero, 0)
        for j in range(zrep):
            pltpu.sync_copy(rows0, acc.at[pl.ds(s * rows_per_tile + j * _CH, _CH)])
        plsc.subcore_barrier()

        def _dstart(t, dbuf, sem):
            r, o = t // _RPC, (t % _RPC) * _CH
            pltpu.async_copy(dst_hbm.at[wid, r, pl.ds(o, _CH)], dbuf, sem)

        def _dwait(t, dbuf, sem):
            r, o = t // _RPC, (t % _RPC) * _CH
            pltpu.make_async_copy(
                dst_hbm.at[wid, r, pl.ds(o, _CH)], dbuf, sem).wait()

        def _gather(t, rbuf, sem):
            r, o = t // _RPC, (t % _RPC) * _CH
            return pltpu.async_copy(
                x_hbm.at[sidx.at[r, pl.ds(o, _CH)]], rbuf, sem)

        def _gwait(t, rbuf, sem):
            r, o = t // _RPC, (t % _RPC) * _CH
            pltpu.make_async_copy(
                x_hbm.at[sidx.at[r, pl.ds(o, _CH)]], rbuf, sem).wait()

        nch = _RPC * nrow
        _dstart(0, d0, semd0)
        _gather(0, rows0, sem0)

        def _pair(i, carry):
            g = 2 * i
            _gather(g + 1, rows1, sem1)
            _dstart(g + 1, d1, semd1)
            _gwait(g, rows0, sem0)
            _dwait(g, d0, semd0)
            pltpu.sync_copy(rows0, acc.at[d0], add=True)

            @pl.when(g + 2 < nch)
            def _():
                _dstart(g + 2, d0, semd0)
                _gather(g + 2, rows0, sem0)

            _gwait(g + 1, rows1, sem1)
            _dwait(g + 1, d1, semd1)
            pltpu.sync_copy(rows1, acc.at[d1], add=True)
            return carry

        lax.fori_loop(0, (_RPC * nrow) // 2, _pair, 0)
        plsc.subcore_barrier()
        pltpu.sync_copy(
            acc.at[pl.ds(s * rows_per_tile, rows_per_tile)],
            out_hbm.at[c, pl.ds(s * rows_per_tile, rows_per_tile)],
        )

    return seg


def _atom_body(x_ref, w_ref, b_ref, o_ref):
    o_ref[...] = jnp.maximum(x_ref[...] @ w_ref[...] + b_ref[...], 0.0)


def _gin_body(x_ref, a0_ref, a1_ref, w_ref, b_ref, o_ref):
    x = x_ref[...]
    t = x + a0_ref[0] + a1_ref[0]
    o_ref[...] = x + jnp.maximum(t @ w_ref[...] + b_ref[...], 0.0)


def _pool_body(x_ref, batch_ref, ng_ref, nb_ref, gw_ref, gb_ref, pw_ref,
               pb_ref, gf_ref, *, nb_graphs):
    x = x_ref[...]
    m = jnp.mean(x, axis=1, keepdims=True)
    v = jnp.mean((x - m) * (x - m), axis=1, keepdims=True)
    xln = (x - m) * lax.rsqrt(v + 1e-5) * ng_ref[...] + nb_ref[...]
    gate = jnp.sum(xln * gw_ref[...], axis=1, keepdims=True) + gb_ref[0, 0]
    xn = jnp.maximum(xln @ pw_ref[...] + pb_ref[...], 0.0)
    onehot = (batch_ref[...] ==
              lax.broadcasted_iota(jnp.int32, (1, nb_graphs), 1))
    oh = onehot.astype(jnp.float32)
    gmax = jnp.max(jnp.where(onehot, gate, -1e30), axis=0, keepdims=True)
    gmax_n = jnp.sum(oh * gmax, axis=1, keepdims=True)
    gexp = jnp.exp(gate - gmax_n)
    gsum = jnp.sum(oh * gexp, axis=0, keepdims=True)
    gsum_n = jnp.sum(oh * gsum, axis=1, keepdims=True)
    att = gexp / (gsum_n + 1e-16)
    gf_ref[...] = lax.dot_general(oh * att, xn, (((0,), (0,)), ((), ())),
                                  preferred_element_type=jnp.float32)


def _tail_body(gf_ref, gx_ref, gw_ref, gb_ref, gg_ref, gbeta_ref, ew_ref,
               eb_ref, h1w_ref, h1b_ref, h2w_ref, h2b_ref, o_ref):
    g = gx_ref[...] @ gw_ref[...] + gb_ref[...]
    m = jnp.mean(g, axis=1, keepdims=True)
    v = jnp.mean((g - m) * (g - m), axis=1, keepdims=True)
    gfeat = jnp.maximum(
        (g - m) * lax.rsqrt(v + 1e-5) * gg_ref[...] + gbeta_ref[...], 0.0)
    gf = gf_ref[...]
    merged = jnp.concatenate([gf, gfeat, gf * gfeat], axis=1)
    emb = merged @ ew_ref[...] + eb_ref[...]
    hid = jnp.maximum(emb @ h1w_ref[...] + h1b_ref[...], 0.0)
    o_ref[...] = jnp.sum(hid * h2w_ref[...], axis=1, keepdims=True) + h2b_ref[0, 0]


def kernel(graph_x, global_x, edge_index, batch, atom_W, atom_b, gin0_W,
           gin0_b, gin1_W, gin1_b, norm_g, norm_b, gate_W, gate_b, pool_W,
           pool_b, glob_W, glob_b, glob_g, glob_beta, emb_W, emb_b, h1_W,
           h1_b, h2_W, h2_b):
    n, ad = graph_x.shape
    h = atom_W.shape[1]
    e = edge_index.shape[1]
    b_graphs, gd = global_x.shape
    nrow = -(-e // (_NW * _IW))
    if (_RPC * nrow) % 2:
        nrow += 1
    epad = _NW * nrow * _IW - e
    npad, _ = _pad_rows(n)

    del epad
    src_all = edge_index[0].astype(jnp.int32)
    dst_all = edge_index[1].astype(jnp.int32)
    slots = nrow * _IW
    base, rem = divmod(e, _NW)
    src_parts, dst_parts = [], []
    off = 0
    for w in range(_NW):
        cw = base + (1 if w < rem else 0)
        pad = slots - cw
        pr = jnp.arange(pad, dtype=jnp.int32)
        src_parts.append(jnp.concatenate([src_all[off:off + cw], pr % n]))
        dst_parts.append(jnp.concatenate(
            [dst_all[off:off + cw], n + (w * 7 + pr) % (npad - n)]))
        off += cw
    src_r = jnp.stack(src_parts).reshape(_NW, nrow, _IW)
    dst_r = jnp.stack(dst_parts).reshape(_NW, nrow, _IW)
    batch_2d = batch.astype(jnp.int32).reshape(n, 1)

    segsum = _make_edge_segsum(n, h, nrow)

    blk = 1000
    grid = n // blk
    row_spec = pl.BlockSpec((blk, h), lambda i: (i, 0))
    full = lambda shape: pl.BlockSpec(shape, lambda i: tuple(0 for _ in shape))

    atom = pl.pallas_call(
        _atom_body,
        grid=(grid,),
        in_specs=[pl.BlockSpec((blk, ad), lambda i: (i, 0)),
                  full((ad, h)), full((1, h))],
        out_specs=row_spec,
        out_shape=jax.ShapeDtypeStruct((n, h), jnp.float32),
    )

    part0 = pl.BlockSpec((1, blk, h), lambda i: (0, i, 0))
    part1 = pl.BlockSpec((1, blk, h), lambda i: (1, i, 0))
    gin = pl.pallas_call(
        _gin_body,
        grid=(grid,),
        in_specs=[row_spec, part0, part1, full((h, h)), full((1, h))],
        out_specs=row_spec,
        out_shape=jax.ShapeDtypeStruct((n, h), jnp.float32),
    )

    pool = pl.pallas_call(
        functools.partial(_pool_body, nb_graphs=b_graphs),
        in_specs=[pl.BlockSpec((n, h), lambda: (0, 0)),
                  pl.BlockSpec((n, 1), lambda: (0, 0)),
                  pl.BlockSpec((1, h), lambda: (0, 0)),
                  pl.BlockSpec((1, h), lambda: (0, 0)),
                  pl.BlockSpec((1, h), lambda: (0, 0)),
                  pl.BlockSpec((1, 1), lambda: (0, 0)),
                  pl.BlockSpec((h, h), lambda: (0, 0)),
                  pl.BlockSpec((1, h), lambda: (0, 0))],
        out_specs=pl.BlockSpec((b_graphs, h), lambda: (0, 0)),
        out_shape=jax.ShapeDtypeStruct((b_graphs, h), jnp.float32),
    )

    tail = pl.pallas_call(
        _tail_body,
        in_specs=[pl.BlockSpec((b_graphs, h), lambda: (0, 0)),
                  pl.BlockSpec((b_graphs, gd), lambda: (0, 0)),
                  pl.BlockSpec((gd, h), lambda: (0, 0)),
                  pl.BlockSpec((1, h), lambda: (0, 0)),
                  pl.BlockSpec((1, h), lambda: (0, 0)),
                  pl.BlockSpec((1, h), lambda: (0, 0)),
                  pl.BlockSpec((3 * h, 64), lambda: (0, 0)),
                  pl.BlockSpec((1, 64), lambda: (0, 0)),
                  pl.BlockSpec((64, 64), lambda: (0, 0)),
                  pl.BlockSpec((1, 64), lambda: (0, 0)),
                  pl.BlockSpec((1, 64), lambda: (0, 0)),
                  pl.BlockSpec((1, 1), lambda: (0, 0))],
        out_specs=pl.BlockSpec((b_graphs, 1), lambda: (0, 0)),
        out_shape=jax.ShapeDtypeStruct((b_graphs, 1), jnp.float32),
    )

    x0 = atom(graph_x, atom_W, atom_b.reshape(1, h))
    p = segsum(x0, src_r, dst_r)
    x1 = gin(x0, p, p, gin0_W, gin0_b.reshape(1, h))
    p = segsum(x1, src_r, dst_r)
    x2 = gin(x1, p, p, gin1_W, gin1_b.reshape(1, h))
    gf = pool(x2, batch_2d, norm_g.reshape(1, h), norm_b.reshape(1, h),
              gate_W.reshape(1, h), gate_b.reshape(1, 1),
              pool_W, pool_b.reshape(1, h))
    return tail(gf, global_x, glob_W, glob_b.reshape(1, h),
                glob_g.reshape(1, h), glob_beta.reshape(1, h),
                emb_W, emb_b.reshape(1, 64), h1_W, h1_b.reshape(1, 64),
                h2_W.reshape(1, 64), h2_b.reshape(1, 1))
